# baseline (speedup 1.0000x reference)
import jax
import jax.numpy as jnp
from jax import lax
from jax.experimental import pallas as pl
from jax.experimental.pallas import tpu as pltpu
from jax.experimental.pallas import tpu_sc as plsc

_BATCH = 4096
_EMB = 64
_DECAY = 1e-4
_NW = 32
_BPW = _BATCH // _NW
_R = 256
_T = _BATCH // _R


def _gather_body(ut, it, users, pos, out_u, out_p,
                 uidx, urows, pidx, prows, sem_u, sem_p):
    wid = lax.axis_index("s") * 2 + lax.axis_index("c")
    base = wid * _BPW
    pltpu.sync_copy(users.at[pl.ds(base, _BPW)], uidx)
    pltpu.sync_copy(pos.at[pl.ds(base, _BPW)], pidx)
    cu = pltpu.async_copy(ut.at[uidx], urows, sem_u)
    cp = pltpu.async_copy(it.at[pidx], prows, sem_p)
    cu.wait()
    cp.wait()
    pltpu.sync_copy(urows, out_u.at[pl.ds(base, _BPW)])
    pltpu.sync_copy(prows, out_p.at[pl.ds(base, _BPW)])


def _loss_body(ug_ref, pg_ref, acc_ref, un_ref, pn_ref):
    t = pl.program_id(0)

    @pl.when(t == 0)
    def _init():
        ug = ug_ref[...]
        pg = pg_ref[...]
        usq = jnp.sum(ug * ug, axis=1, keepdims=True)
        psq = jnp.sum(pg * pg, axis=1, keepdims=True)
        un = ug / jnp.sqrt(usq)
        pn = pg / jnp.sqrt(psq)
        un_ref[...] = un.astype(jnp.bfloat16)
        pn_ref[...] = pn.astype(jnp.bfloat16)
        acc_ref[0] = jnp.sum((un - pn) ** 2)
        acc_ref[1] = jnp.sum(usq) + jnp.sum(psq)
        acc_ref[2] = jnp.float32(-_BATCH)
        acc_ref[3] = jnp.float32(-_BATCH)

    dn = (((1,), (1,)), ((), ()))
    gu = lax.dot_general(un_ref[pl.ds(t * _R, _R), :], un_ref[...], dn,
                         preferred_element_type=jnp.float32)
    gp = lax.dot_general(pn_ref[pl.ds(t * _R, _R), :], pn_ref[...], dn,
                         preferred_element_type=jnp.float32)
    acc_ref[2] += jnp.sum(jnp.exp(jnp.minimum(4.0 * gu - 4.0, 0.0)))
    acc_ref[3] += jnp.sum(jnp.exp(jnp.minimum(4.0 * gp - 4.0, 0.0)))


def kernel(user_embed, item_embed, users, pos_items):
    gather = pl.kernel(
        _gather_body,
        mesh=plsc.VectorSubcoreMesh(core_axis_name="c", subcore_axis_name="s"),
        out_type=[jax.ShapeDtypeStruct((_BATCH, _EMB), jnp.float32),
                  jax.ShapeDtypeStruct((_BATCH, _EMB), jnp.float32)],
        scratch_types=[
            pltpu.VMEM((_BPW,), jnp.int32),
            pltpu.VMEM((_BPW, _EMB), jnp.float32),
            pltpu.VMEM((_BPW,), jnp.int32),
            pltpu.VMEM((_BPW, _EMB), jnp.float32),
            pltpu.SemaphoreType.DMA,
            pltpu.SemaphoreType.DMA,
        ],
        compiler_params=pltpu.CompilerParams(use_tc_tiling_on_sc=False),
    )
    ug, pg = gather(user_embed, item_embed, users, pos_items)

    acc = pl.pallas_call(
        _loss_body,
        grid=(_T,),
        in_specs=[pl.BlockSpec((_BATCH, _EMB), lambda t: (0, 0)),
                  pl.BlockSpec((_BATCH, _EMB), lambda t: (0, 0))],
        out_specs=pl.BlockSpec((4,), lambda t: (0,), memory_space=pltpu.SMEM),
        out_shape=jax.ShapeDtypeStruct((4,), jnp.float32),
        scratch_shapes=[
            pltpu.VMEM((_BATCH, _EMB), jnp.bfloat16),
            pltpu.VMEM((_BATCH, _EMB), jnp.bfloat16),
        ],
    )(ug, pg)

    n_pairs = _BATCH * (_BATCH - 1) / 2.0
    align = acc[0] / _BATCH
    uniformity = 0.5 * (jnp.log(acc[2] * (0.5 / n_pairs))
                        + jnp.log(acc[3] * (0.5 / n_pairs)))
    emb_loss = (_DECAY * 0.5 / _BATCH) * acc[1]
    return align + uniformity + emb_loss, emb_loss
